# baseline (speedup 1.0000x reference)
import functools

import jax
import jax.numpy as jnp
from jax.experimental import pallas as pl
from jax.experimental.pallas import tpu as pltpu

_ALPHA = 0.1
_K = 2
_BM = 1024
_BK = 1024


def _hop1_kernel(adj_ref, w_ref, d_ref, s1_ref, abf_ref, *, nbi, nbj, n):
    i = pl.program_id(0)
    j = pl.program_id(1)

    a = adj_ref[...] * w_ref[...]

    def _mask_cols(x):
        col = jax.lax.broadcasted_iota(jnp.int32, x.shape, 1)
        return jnp.where(col < n - j * _BK, x, 0.0)

    def _mask_rows(x):
        row = jax.lax.broadcasted_iota(jnp.int32, x.shape, 0)
        return jnp.where(row < n - i * _BM, x, 0.0)

    a = jax.lax.cond(j == nbj - 1, _mask_cols, lambda x: x, a)
    a = jax.lax.cond(i == nbi - 1, _mask_rows, lambda x: x, a)

    a_bf = a.astype(jnp.bfloat16)
    abf_ref[...] = a_bf

    @pl.when(j == 0)
    def _():
        s1_ref[...] = jnp.zeros_like(s1_ref)

    s1_ref[...] += jnp.dot(a_bf, d_ref[...], preferred_element_type=jnp.float32)


def _hop2_kernel(abf_ref, s1b_ref, s1_ref, d_ref, w_ref, out_ref, acc_ref, *, nbj):
    j = pl.program_id(1)

    @pl.when(j == 0)
    def _():
        acc_ref[...] = jnp.zeros_like(acc_ref)

    acc_ref[...] += jnp.dot(abf_ref[...], s1b_ref[...], preferred_element_type=jnp.float32)

    @pl.when(j == nbj - 1)
    def _():
        t = _ALPHA * d_ref[...] + ((1.0 - _ALPHA) / _K) * (s1_ref[...] + acc_ref[...])
        out_ref[...] = jnp.maximum(
            jnp.dot(t, w_ref[...], preferred_element_type=jnp.float32), 0.0
        )


def kernel(adjacency_matrices, weights_matrices, data, W):
    n, c = data.shape
    f = W.shape[1]
    nbi = pl.cdiv(n, _BM)
    nbj = pl.cdiv(n, _BK)
    n_pad = nbi * _BM

    data_p = jnp.zeros((n_pad, c), jnp.float32).at[:n].set(data)
    data_bf = data_p.astype(jnp.bfloat16)

    s1, a_bf = pl.pallas_call(
        functools.partial(_hop1_kernel, nbi=nbi, nbj=nbj, n=n),
        grid=(nbi, nbj),
        in_specs=[
            pl.BlockSpec((_BM, _BK), lambda i, j: (i, j)),
            pl.BlockSpec((_BM, _BK), lambda i, j: (i, j)),
            pl.BlockSpec((_BK, c), lambda i, j: (j, 0)),
        ],
        out_specs=[
            pl.BlockSpec((_BM, c), lambda i, j: (i, 0)),
            pl.BlockSpec((_BM, _BK), lambda i, j: (i, j)),
        ],
        out_shape=[
            jax.ShapeDtypeStruct((n_pad, c), jnp.float32),
            jax.ShapeDtypeStruct((n_pad, n_pad), jnp.bfloat16),
        ],
        compiler_params=pltpu.CompilerParams(
            dimension_semantics=("arbitrary", "arbitrary"),
        ),
    )(adjacency_matrices, weights_matrices, data_bf)

    s1_bf = s1.astype(jnp.bfloat16)

    out = pl.pallas_call(
        functools.partial(_hop2_kernel, nbj=nbj),
        grid=(nbi, nbj),
        in_specs=[
            pl.BlockSpec((_BM, _BK), lambda i, j: (i, j)),
            pl.BlockSpec((_BK, c), lambda i, j: (j, 0)),
            pl.BlockSpec((_BM, c), lambda i, j: (i, 0)),
            pl.BlockSpec((_BM, c), lambda i, j: (i, 0)),
            pl.BlockSpec((c, f), lambda i, j: (0, 0)),
        ],
        out_specs=pl.BlockSpec((_BM, f), lambda i, j: (i, 0)),
        out_shape=jax.ShapeDtypeStruct((n, f), jnp.float32),
        scratch_shapes=[pltpu.VMEM((_BM, c), jnp.float32)],
        compiler_params=pltpu.CompilerParams(
            dimension_semantics=("arbitrary", "arbitrary"),
        ),
    )(a_bf, s1_bf, s1, data_p, W)

    return out


# speedup vs baseline: 1.2244x; 1.2244x over previous
import functools

import jax
import jax.numpy as jnp
from jax.experimental import pallas as pl
from jax.experimental.pallas import tpu as pltpu

_ALPHA = 0.1
_K = 2
_BM = 1024
_BK = 1024


def _hop1_kernel(adj_ref, w_ref, d_ref, s1_ref, abf_ref, *, nbj, n):
    j = pl.program_id(1)

    a = adj_ref[...] * w_ref[...]

    col = jax.lax.broadcasted_iota(jnp.int32, a.shape, 1)
    a_bf = jnp.where(col < n - j * _BK, a, 0.0).astype(jnp.bfloat16)
    abf_ref[...] = a_bf

    @pl.when(j == 0)
    def _():
        s1_ref[...] = jnp.zeros_like(s1_ref)

    s1_ref[...] += jnp.dot(a_bf, d_ref[...], preferred_element_type=jnp.float32)


def _hop2_kernel(abf_ref, s1b_ref, s1_ref, d_ref, w_ref, out_ref, acc_ref, *, nbj):
    j = pl.program_id(1)

    @pl.when(j == 0)
    def _():
        acc_ref[...] = jnp.zeros_like(acc_ref)

    acc_ref[...] += jnp.dot(abf_ref[...], s1b_ref[...], preferred_element_type=jnp.float32)

    @pl.when(j == nbj - 1)
    def _():
        t = _ALPHA * d_ref[...] + ((1.0 - _ALPHA) / _K) * (s1_ref[...] + acc_ref[...])
        out_ref[...] = jnp.maximum(
            jnp.dot(t, w_ref[...], preferred_element_type=jnp.float32), 0.0
        )


def kernel(adjacency_matrices, weights_matrices, data, W):
    n, c = data.shape
    f = W.shape[1]
    nbi = pl.cdiv(n, _BM)
    nbj = pl.cdiv(n, _BK)
    n_pad = nbi * _BM

    data_p = jnp.zeros((n_pad, c), jnp.float32).at[:n].set(data)
    data_bf = data_p.astype(jnp.bfloat16)

    s1, a_bf = pl.pallas_call(
        functools.partial(_hop1_kernel, nbj=nbj, n=n),
        grid=(nbi, nbj),
        in_specs=[
            pl.BlockSpec((_BM, _BK), lambda i, j: (i, j)),
            pl.BlockSpec((_BM, _BK), lambda i, j: (i, j)),
            pl.BlockSpec((_BK, c), lambda i, j: (j, 0)),
        ],
        out_specs=[
            pl.BlockSpec((_BM, c), lambda i, j: (i, 0)),
            pl.BlockSpec((_BM, _BK), lambda i, j: (i, j)),
        ],
        out_shape=[
            jax.ShapeDtypeStruct((n_pad, c), jnp.float32),
            jax.ShapeDtypeStruct((n_pad, n_pad), jnp.bfloat16),
        ],
        compiler_params=pltpu.CompilerParams(
            dimension_semantics=("parallel", "arbitrary"),
        ),
    )(adjacency_matrices, weights_matrices, data_bf)

    row_ok = (jnp.arange(n_pad) < n)[:, None]
    s1_bf = jnp.where(row_ok, s1, 0.0).astype(jnp.bfloat16)

    out = pl.pallas_call(
        functools.partial(_hop2_kernel, nbj=nbj),
        grid=(nbi, nbj),
        in_specs=[
            pl.BlockSpec((_BM, _BK), lambda i, j: (i, j)),
            pl.BlockSpec((_BK, c), lambda i, j: (j, 0)),
            pl.BlockSpec((_BM, c), lambda i, j: (i, 0)),
            pl.BlockSpec((_BM, c), lambda i, j: (i, 0)),
            pl.BlockSpec((c, f), lambda i, j: (0, 0)),
        ],
        out_specs=pl.BlockSpec((_BM, f), lambda i, j: (i, 0)),
        out_shape=jax.ShapeDtypeStruct((n, f), jnp.float32),
        scratch_shapes=[pltpu.VMEM((_BM, c), jnp.float32)],
        compiler_params=pltpu.CompilerParams(
            dimension_semantics=("parallel", "arbitrary"),
        ),
    )(a_bf, s1_bf, s1, data_p, W)

    return out


# speedup vs baseline: 1.4016x; 1.1447x over previous
import functools

import jax
import jax.numpy as jnp
from jax.experimental import pallas as pl
from jax.experimental.pallas import tpu as pltpu

_ALPHA = 0.1
_K = 2
_BM = 1024
_BK = 1024


_A_DTYPE = jnp.float8_e4m3fn


def _hop1_kernel(adj_ref, w_ref, d_ref, s1_ref, s1c_ref, ac_ref, *, nbi, nbj, n):
    i = pl.program_id(0)
    j = pl.program_id(1)

    a = adj_ref[...] * w_ref[...]

    col = jax.lax.broadcasted_iota(jnp.int32, a.shape, 1)
    a_m = jnp.where(col < n - j * _BK, a, 0.0)
    ac_ref[...] = a_m.astype(_A_DTYPE)

    @pl.when(j == 0)
    def _():
        s1_ref[...] = jnp.zeros_like(s1_ref)

    s1_ref[...] += jnp.dot(
        a_m.astype(jnp.bfloat16), d_ref[...], preferred_element_type=jnp.float32
    )

    @pl.when(j == nbj - 1)
    def _():
        row = jax.lax.broadcasted_iota(jnp.int32, s1_ref.shape, 0)
        s1c_ref[...] = jnp.where(row < n - i * _BM, s1_ref[...], 0.0).astype(
            jnp.bfloat16
        )


def _hop2_kernel(ac_ref, s1b_ref, s1_ref, d_ref, w_ref, out_ref, acc_ref, *, nbj):
    j = pl.program_id(1)

    @pl.when(j == 0)
    def _():
        acc_ref[...] = jnp.zeros_like(acc_ref)

    acc_ref[...] += jnp.dot(
        ac_ref[...].astype(jnp.bfloat16), s1b_ref[...], preferred_element_type=jnp.float32
    )

    @pl.when(j == nbj - 1)
    def _():
        t = _ALPHA * d_ref[...] + ((1.0 - _ALPHA) / _K) * (s1_ref[...] + acc_ref[...])
        out_ref[...] = jnp.maximum(
            jnp.dot(t, w_ref[...], preferred_element_type=jnp.float32), 0.0
        )


def kernel(adjacency_matrices, weights_matrices, data, W):
    n, c = data.shape
    f = W.shape[1]
    nbi = pl.cdiv(n, _BM)
    nbj = pl.cdiv(n, _BK)
    n_pad = nbi * _BM

    data_p = jnp.zeros((n_pad, c), jnp.float32).at[:n].set(data)
    data_bf = data_p.astype(jnp.bfloat16)

    s1, s1_bf, a_c = pl.pallas_call(
        functools.partial(_hop1_kernel, nbi=nbi, nbj=nbj, n=n),
        grid=(nbi, nbj),
        in_specs=[
            pl.BlockSpec((_BM, _BK), lambda i, j: (i, j)),
            pl.BlockSpec((_BM, _BK), lambda i, j: (i, j)),
            pl.BlockSpec((_BK, c), lambda i, j: (j, 0)),
        ],
        out_specs=[
            pl.BlockSpec((_BM, c), lambda i, j: (i, 0)),
            pl.BlockSpec((_BM, c), lambda i, j: (i, 0)),
            pl.BlockSpec((_BM, _BK), lambda i, j: (i, j)),
        ],
        out_shape=[
            jax.ShapeDtypeStruct((n_pad, c), jnp.float32),
            jax.ShapeDtypeStruct((n_pad, c), jnp.bfloat16),
            jax.ShapeDtypeStruct((n_pad, n_pad), _A_DTYPE),
        ],
        compiler_params=pltpu.CompilerParams(
            dimension_semantics=("parallel", "arbitrary"),
        ),
    )(adjacency_matrices, weights_matrices, data_bf)

    out = pl.pallas_call(
        functools.partial(_hop2_kernel, nbj=nbj),
        grid=(nbi, nbj),
        in_specs=[
            pl.BlockSpec((_BM, _BK), lambda i, j: (i, j)),
            pl.BlockSpec((_BK, c), lambda i, j: (j, 0)),
            pl.BlockSpec((_BM, c), lambda i, j: (i, 0)),
            pl.BlockSpec((_BM, c), lambda i, j: (i, 0)),
            pl.BlockSpec((c, f), lambda i, j: (0, 0)),
        ],
        out_specs=pl.BlockSpec((_BM, f), lambda i, j: (i, 0)),
        out_shape=jax.ShapeDtypeStruct((n, f), jnp.float32),
        scratch_shapes=[pltpu.VMEM((_BM, c), jnp.float32)],
        compiler_params=pltpu.CompilerParams(
            dimension_semantics=("parallel", "arbitrary"),
        ),
    )(a_c, s1_bf, s1, data_p, W)

    return out


# speedup vs baseline: 1.4594x; 1.0413x over previous
import functools

import jax
import jax.numpy as jnp
from jax.experimental import pallas as pl
from jax.experimental.pallas import tpu as pltpu

_ALPHA = 0.1
_K = 2
_BM = 1024
_BK = 1024


_A_DTYPE = jnp.float8_e4m3fn


def _hop1_kernel(adj_ref, w_ref, d_ref, s1_ref, s1c_ref, ac_ref, *, nbi, nbj, n):
    i = pl.program_id(0)
    j = pl.program_id(1)

    a = adj_ref[...] * w_ref[...]

    col = jax.lax.broadcasted_iota(jnp.int32, a.shape, 1)
    a_m = jnp.where(col < n - j * _BK, a, 0.0)
    ac_ref[...] = a_m.astype(_A_DTYPE)

    @pl.when(j == 0)
    def _():
        s1_ref[...] = jnp.zeros_like(s1_ref)

    s1_ref[...] += jnp.dot(
        a_m.astype(jnp.bfloat16), d_ref[...], preferred_element_type=jnp.float32
    )

    @pl.when(j == nbj - 1)
    def _():
        row = jax.lax.broadcasted_iota(jnp.int32, s1_ref.shape, 0)
        s1c_ref[...] = jnp.where(row < n - i * _BM, s1_ref[...], 0.0).astype(
            s1c_ref.dtype
        )


def _hop2_kernel(ac_ref, s1b_ref, s1_ref, d_ref, w_ref, out_ref, acc_ref, *, nbj):
    j = pl.program_id(1)

    @pl.when(j == 0)
    def _():
        acc_ref[...] = jnp.zeros_like(acc_ref)

    acc_ref[...] += jnp.dot(
        ac_ref[...], s1b_ref[...], preferred_element_type=jnp.float32
    )

    @pl.when(j == nbj - 1)
    def _():
        t = _ALPHA * d_ref[...] + ((1.0 - _ALPHA) / _K) * (s1_ref[...] + acc_ref[...])
        out_ref[...] = jnp.maximum(
            jnp.dot(t, w_ref[...], preferred_element_type=jnp.float32), 0.0
        )


def kernel(adjacency_matrices, weights_matrices, data, W):
    n, c = data.shape
    f = W.shape[1]
    nbi = pl.cdiv(n, _BM)
    nbj = pl.cdiv(n, _BK)
    n_pad = nbi * _BM

    data_p = jnp.zeros((n_pad, c), jnp.float32).at[:n].set(data)
    data_bf = data_p.astype(jnp.bfloat16)

    s1, s1_bf, a_c = pl.pallas_call(
        functools.partial(_hop1_kernel, nbi=nbi, nbj=nbj, n=n),
        grid=(nbi, nbj),
        in_specs=[
            pl.BlockSpec((_BM, _BK), lambda i, j: (i, j)),
            pl.BlockSpec((_BM, _BK), lambda i, j: (i, j)),
            pl.BlockSpec((_BK, c), lambda i, j: (j, 0)),
        ],
        out_specs=[
            pl.BlockSpec((_BM, c), lambda i, j: (i, 0)),
            pl.BlockSpec((_BM, c), lambda i, j: (i, 0)),
            pl.BlockSpec((_BM, _BK), lambda i, j: (i, j)),
        ],
        out_shape=[
            jax.ShapeDtypeStruct((n_pad, c), jnp.float32),
            jax.ShapeDtypeStruct((n_pad, c), _A_DTYPE),
            jax.ShapeDtypeStruct((n_pad, n_pad), _A_DTYPE),
        ],
        compiler_params=pltpu.CompilerParams(
            dimension_semantics=("parallel", "arbitrary"),
        ),
    )(adjacency_matrices, weights_matrices, data_bf)

    out = pl.pallas_call(
        functools.partial(_hop2_kernel, nbj=nbj),
        grid=(nbi, nbj),
        in_specs=[
            pl.BlockSpec((_BM, _BK), lambda i, j: (i, j)),
            pl.BlockSpec((_BK, c), lambda i, j: (j, 0)),
            pl.BlockSpec((_BM, c), lambda i, j: (i, 0)),
            pl.BlockSpec((_BM, c), lambda i, j: (i, 0)),
            pl.BlockSpec((c, f), lambda i, j: (0, 0)),
        ],
        out_specs=pl.BlockSpec((_BM, f), lambda i, j: (i, 0)),
        out_shape=jax.ShapeDtypeStruct((n, f), jnp.float32),
        scratch_shapes=[pltpu.VMEM((_BM, c), jnp.float32)],
        compiler_params=pltpu.CompilerParams(
            dimension_semantics=("parallel", "arbitrary"),
        ),
    )(a_c, s1_bf, s1, data_p, W)

    return out


# speedup vs baseline: 1.5880x; 1.0881x over previous
import functools

import jax
import jax.numpy as jnp
from jax.experimental import pallas as pl
from jax.experimental.pallas import tpu as pltpu

_ALPHA = 0.1
_K = 2
_BM = 1024
_BK = 2048


_A_DTYPE = jnp.float8_e4m3fn


def _hop1_kernel(adj_ref, w_ref, d_ref, s1_ref, s1c_ref, ac_ref, *, nbi, nbj, n):
    i = pl.program_id(0)
    j = pl.program_id(1)

    a = adj_ref[...] * w_ref[...]

    col = jax.lax.broadcasted_iota(jnp.int32, a.shape, 1)
    a_m = jnp.where(col < n - j * _BK, a, 0.0)
    ac_ref[...] = a_m.astype(_A_DTYPE)

    @pl.when(j == 0)
    def _():
        s1_ref[...] = jnp.zeros_like(s1_ref)

    s1_ref[...] += jnp.dot(
        a_m.astype(jnp.bfloat16), d_ref[...], preferred_element_type=jnp.float32
    )

    @pl.when(j == nbj - 1)
    def _():
        row = jax.lax.broadcasted_iota(jnp.int32, s1_ref.shape, 0)
        s1c_ref[...] = jnp.where(row < n - i * _BM, s1_ref[...], 0.0).astype(
            s1c_ref.dtype
        )


def _hop2_kernel(ac_ref, s1b_ref, s1_ref, d_ref, w_ref, out_ref, acc_ref, *, nbj):
    j = pl.program_id(1)

    @pl.when(j == 0)
    def _():
        acc_ref[...] = jnp.zeros_like(acc_ref)

    acc_ref[...] += jnp.dot(
        ac_ref[...], s1b_ref[...], preferred_element_type=jnp.float32
    )

    @pl.when(j == nbj - 1)
    def _():
        t = _ALPHA * d_ref[...] + ((1.0 - _ALPHA) / _K) * (s1_ref[...] + acc_ref[...])
        out_ref[...] = jnp.maximum(
            jnp.dot(t, w_ref[...], preferred_element_type=jnp.float32), 0.0
        )


def kernel(adjacency_matrices, weights_matrices, data, W):
    n, c = data.shape
    f = W.shape[1]
    nbi = pl.cdiv(n, _BM)
    nbj = pl.cdiv(n, _BK)
    n_pad = nbi * _BM

    data_p = jnp.zeros((n_pad, c), jnp.float32).at[:n].set(data)
    data_bf = data_p.astype(jnp.bfloat16)

    s1, s1_bf, a_c = pl.pallas_call(
        functools.partial(_hop1_kernel, nbi=nbi, nbj=nbj, n=n),
        grid=(nbi, nbj),
        in_specs=[
            pl.BlockSpec((_BM, _BK), lambda i, j: (i, j)),
            pl.BlockSpec((_BM, _BK), lambda i, j: (i, j)),
            pl.BlockSpec((_BK, c), lambda i, j: (j, 0)),
        ],
        out_specs=[
            pl.BlockSpec((_BM, c), lambda i, j: (i, 0)),
            pl.BlockSpec((_BM, c), lambda i, j: (i, 0)),
            pl.BlockSpec((_BM, _BK), lambda i, j: (i, j)),
        ],
        out_shape=[
            jax.ShapeDtypeStruct((n_pad, c), jnp.float32),
            jax.ShapeDtypeStruct((n_pad, c), _A_DTYPE),
            jax.ShapeDtypeStruct((n_pad, n_pad), _A_DTYPE),
        ],
        compiler_params=pltpu.CompilerParams(
            dimension_semantics=("parallel", "arbitrary"),
        ),
    )(adjacency_matrices, weights_matrices, data_bf)

    out = pl.pallas_call(
        functools.partial(_hop2_kernel, nbj=nbj),
        grid=(nbi, nbj),
        in_specs=[
            pl.BlockSpec((_BM, _BK), lambda i, j: (i, j)),
            pl.BlockSpec((_BK, c), lambda i, j: (j, 0)),
            pl.BlockSpec((_BM, c), lambda i, j: (i, 0)),
            pl.BlockSpec((_BM, c), lambda i, j: (i, 0)),
            pl.BlockSpec((c, f), lambda i, j: (0, 0)),
        ],
        out_specs=pl.BlockSpec((_BM, f), lambda i, j: (i, 0)),
        out_shape=jax.ShapeDtypeStruct((n, f), jnp.float32),
        scratch_shapes=[pltpu.VMEM((_BM, c), jnp.float32)],
        compiler_params=pltpu.CompilerParams(
            dimension_semantics=("parallel", "arbitrary"),
        ),
    )(a_c, s1_bf, s1, data_p, W)

    return out


# speedup vs baseline: 1.6143x; 1.0166x over previous
import functools

import jax
import jax.numpy as jnp
from jax.experimental import pallas as pl
from jax.experimental.pallas import tpu as pltpu

_ALPHA = 0.1
_K = 2
_BM = 1024
_BK = 2560


_A_DTYPE = jnp.float8_e4m3fn


def _hop1_kernel(adj_ref, w_ref, d_ref, s1_ref, s1c_ref, ac_ref, *, nbi, nbj, n):
    i = pl.program_id(0)
    j = pl.program_id(1)

    a = adj_ref[...] * w_ref[...]

    col = jax.lax.broadcasted_iota(jnp.int32, a.shape, 1)
    a_m = jnp.where(col < n - j * _BK, a, 0.0)
    ac_ref[...] = a_m.astype(_A_DTYPE)

    @pl.when(j == 0)
    def _():
        s1_ref[...] = jnp.zeros_like(s1_ref)

    s1_ref[...] += jnp.dot(
        a_m.astype(jnp.bfloat16), d_ref[...], preferred_element_type=jnp.float32
    )

    @pl.when(j == nbj - 1)
    def _():
        row = jax.lax.broadcasted_iota(jnp.int32, s1_ref.shape, 0)
        s1c_ref[...] = jnp.where(row < n - i * _BM, s1_ref[...], 0.0).astype(
            s1c_ref.dtype
        )


def _hop2_kernel(ac_ref, s1b_ref, s1_ref, d_ref, w_ref, out_ref, acc_ref, *, nbj):
    j = pl.program_id(1)

    @pl.when(j == 0)
    def _():
        acc_ref[...] = jnp.zeros_like(acc_ref)

    acc_ref[...] += jnp.dot(
        ac_ref[...], s1b_ref[...], preferred_element_type=jnp.float32
    )

    @pl.when(j == nbj - 1)
    def _():
        t = _ALPHA * d_ref[...] + ((1.0 - _ALPHA) / _K) * (s1_ref[...] + acc_ref[...])
        out_ref[...] = jnp.maximum(
            jnp.dot(t, w_ref[...], preferred_element_type=jnp.float32), 0.0
        )


def kernel(adjacency_matrices, weights_matrices, data, W):
    n, c = data.shape
    f = W.shape[1]
    nbi = pl.cdiv(n, _BM)
    nbj = pl.cdiv(n, _BK)
    n_pad = nbi * _BM

    data_p = jnp.zeros((n_pad, c), jnp.float32).at[:n].set(data)
    data_bf = data_p.astype(jnp.bfloat16)

    s1, s1_bf, a_c = pl.pallas_call(
        functools.partial(_hop1_kernel, nbi=nbi, nbj=nbj, n=n),
        grid=(nbi, nbj),
        in_specs=[
            pl.BlockSpec((_BM, _BK), lambda i, j: (i, j)),
            pl.BlockSpec((_BM, _BK), lambda i, j: (i, j)),
            pl.BlockSpec((_BK, c), lambda i, j: (j, 0)),
        ],
        out_specs=[
            pl.BlockSpec((_BM, c), lambda i, j: (i, 0)),
            pl.BlockSpec((_BM, c), lambda i, j: (i, 0)),
            pl.BlockSpec((_BM, _BK), lambda i, j: (i, j)),
        ],
        out_shape=[
            jax.ShapeDtypeStruct((n_pad, c), jnp.float32),
            jax.ShapeDtypeStruct((n_pad, c), _A_DTYPE),
            jax.ShapeDtypeStruct((n_pad, n_pad), _A_DTYPE),
        ],
        compiler_params=pltpu.CompilerParams(
            dimension_semantics=("parallel", "arbitrary"),
        ),
    )(adjacency_matrices, weights_matrices, data_bf)

    out = pl.pallas_call(
        functools.partial(_hop2_kernel, nbj=nbj),
        grid=(nbi, nbj),
        in_specs=[
            pl.BlockSpec((_BM, _BK), lambda i, j: (i, j)),
            pl.BlockSpec((_BK, c), lambda i, j: (j, 0)),
            pl.BlockSpec((_BM, c), lambda i, j: (i, 0)),
            pl.BlockSpec((_BM, c), lambda i, j: (i, 0)),
            pl.BlockSpec((c, f), lambda i, j: (0, 0)),
        ],
        out_specs=pl.BlockSpec((_BM, f), lambda i, j: (i, 0)),
        out_shape=jax.ShapeDtypeStruct((n, f), jnp.float32),
        scratch_shapes=[pltpu.VMEM((_BM, c), jnp.float32)],
        compiler_params=pltpu.CompilerParams(
            dimension_semantics=("parallel", "arbitrary"),
        ),
    )(a_c, s1_bf, s1, data_p, W)

    return out


# speedup vs baseline: 1.6849x; 1.0437x over previous
import functools

import jax
import jax.numpy as jnp
import numpy as np
from jax.experimental import pallas as pl
from jax.experimental.pallas import tpu as pltpu

_ALPHA = 0.1
_K = 2
_BM = 1024
_BK = 2048

_A_DTYPE = jnp.float8_e4m3fn


def _hop1_kernel(
    adj_ref, w_ref, d_ref, s1_ref, s1c_ref, ac_ref, o2_ref, s1c_all, acc2_ref,
    *, nbi, nbj, n,
):
    i = pl.program_id(0)
    j = pl.program_id(1)

    a = adj_ref[...] * w_ref[...]

    col = jax.lax.broadcasted_iota(jnp.int32, a.shape, 1)
    a_m = jnp.where(col < n - j * _BK, a, 0.0)
    a_c = a_m.astype(_A_DTYPE)
    ac_ref[...] = a_c

    drow = jax.lax.broadcasted_iota(jnp.int32, d_ref.shape, 0)
    d = jnp.where(drow < n - j * _BK, d_ref[...], 0.0).astype(jnp.bfloat16)

    @pl.when(j == 0)
    def _():
        s1_ref[...] = jnp.zeros_like(s1_ref)
        acc2_ref[...] = jnp.zeros_like(acc2_ref)

    s1_ref[...] += jnp.dot(
        a_m.astype(jnp.bfloat16), d, preferred_element_type=jnp.float32
    )

    @pl.when((j + 1) * _BK <= i * _BM)
    def _():
        s1c_blk = s1c_all[pl.ds(j * _BK, _BK), :]
        acc2_ref[...] += jnp.dot(a_c, s1c_blk, preferred_element_type=jnp.float32)

    @pl.when(j == nbj - 1)
    def _():
        row = jax.lax.broadcasted_iota(jnp.int32, s1_ref.shape, 0)
        s1c_blk = jnp.where(row < n - i * _BM, s1_ref[...], 0.0).astype(_A_DTYPE)
        s1c_ref[...] = s1c_blk
        s1c_all[pl.ds(i * _BM, _BM), :] = s1c_blk
        o2_ref[...] = acc2_ref[...]


def _hop2_kernel(
    i_arr, j_arr, first_arr, ac_ref, s1b_ref, o2_ref, s1_ref, d_ref, w_ref,
    out_ref, acc_ref, *, nbj,
):
    t = pl.program_id(0)

    @pl.when(first_arr[t] == 1)
    def _():
        acc_ref[...] = o2_ref[...]

    acc_ref[...] += jnp.dot(
        ac_ref[...], s1b_ref[...], preferred_element_type=jnp.float32
    )

    @pl.when(j_arr[t] == nbj - 1)
    def _():
        t_val = _ALPHA * d_ref[...] + ((1.0 - _ALPHA) / _K) * (
            s1_ref[...] + acc_ref[...]
        )
        out_ref[...] = jnp.maximum(
            jnp.dot(t_val, w_ref[...], preferred_element_type=jnp.float32), 0.0
        )


def kernel(adjacency_matrices, weights_matrices, data, W):
    n, c = data.shape
    f = W.shape[1]
    nbi = pl.cdiv(n, _BM)
    nbj = pl.cdiv(n, _BK)
    npr = nbi * _BM
    npc = nbj * _BK

    s1, s1c, a_c, o2 = pl.pallas_call(
        functools.partial(_hop1_kernel, nbi=nbi, nbj=nbj, n=n),
        grid=(nbi, nbj),
        in_specs=[
            pl.BlockSpec((_BM, _BK), lambda i, j: (i, j)),
            pl.BlockSpec((_BM, _BK), lambda i, j: (i, j)),
            pl.BlockSpec((_BK, c), lambda i, j: (j, 0)),
        ],
        out_specs=[
            pl.BlockSpec((_BM, c), lambda i, j: (i, 0)),
            pl.BlockSpec((_BM, c), lambda i, j: (i, 0)),
            pl.BlockSpec((_BM, _BK), lambda i, j: (i, j)),
            pl.BlockSpec((_BM, c), lambda i, j: (i, 0)),
        ],
        out_shape=[
            jax.ShapeDtypeStruct((npr, c), jnp.float32),
            jax.ShapeDtypeStruct((npr, c), _A_DTYPE),
            jax.ShapeDtypeStruct((npr, npc), _A_DTYPE),
            jax.ShapeDtypeStruct((npr, c), jnp.float32),
        ],
        scratch_shapes=[
            pltpu.VMEM((npc, c), _A_DTYPE),
            pltpu.VMEM((_BM, c), jnp.float32),
        ],
        compiler_params=pltpu.CompilerParams(
            dimension_semantics=("arbitrary", "arbitrary"),
        ),
    )(adjacency_matrices, weights_matrices, data)

    tiles = [
        (i, j)
        for i in range(nbi)
        for j in range(nbj)
        if (j + 1) * _BK > i * _BM
    ]
    i_arr = np.array([t[0] for t in tiles], np.int32)
    j_arr = np.array([t[1] for t in tiles], np.int32)
    first_arr = np.array(
        [1 if (k == 0 or tiles[k][0] != tiles[k - 1][0]) else 0 for k in range(len(tiles))],
        np.int32,
    )

    out = pl.pallas_call(
        functools.partial(_hop2_kernel, nbj=nbj),
        grid_spec=pltpu.PrefetchScalarGridSpec(
            num_scalar_prefetch=3,
            grid=(len(tiles),),
            in_specs=[
                pl.BlockSpec((_BM, _BK), lambda t, ia, ja, fa: (ia[t], ja[t])),
                pl.BlockSpec((_BK, c), lambda t, ia, ja, fa: (ja[t], 0)),
                pl.BlockSpec((_BM, c), lambda t, ia, ja, fa: (ia[t], 0)),
                pl.BlockSpec((_BM, c), lambda t, ia, ja, fa: (ia[t], 0)),
                pl.BlockSpec((_BM, c), lambda t, ia, ja, fa: (ia[t], 0)),
                pl.BlockSpec((c, f), lambda t, ia, ja, fa: (0, 0)),
            ],
            out_specs=pl.BlockSpec((_BM, f), lambda t, ia, ja, fa: (ia[t], 0)),
            scratch_shapes=[pltpu.VMEM((_BM, c), jnp.float32)],
        ),
        out_shape=jax.ShapeDtypeStruct((n, f), jnp.float32),
        compiler_params=pltpu.CompilerParams(
            dimension_semantics=("arbitrary",),
        ),
    )(i_arr, j_arr, first_arr, a_c, s1c, o2, s1, data, W)

    return out


# speedup vs baseline: 1.6973x; 1.0074x over previous
import functools

import jax
import jax.numpy as jnp
import numpy as np
from jax.experimental import pallas as pl
from jax.experimental.pallas import tpu as pltpu

_ALPHA = 0.1
_K = 2
_BM = 1024
_BK = 2048

_A_DTYPE = jnp.float8_e4m3fn


def _hop1_kernel(
    adj_ref, w_ref, d_ref, s1_ref, s1c_ref, ac_ref, o2_ref, s1c_all, acc2_ref,
    *, nbi, nbj, n,
):
    i = pl.program_id(0)
    j = pl.program_id(1)

    a = adj_ref[...] * w_ref[...]

    col = jax.lax.broadcasted_iota(jnp.int32, a.shape, 1)
    a_m = jnp.where(col < n - j * _BK, a, 0.0)
    a_c = a_m.astype(_A_DTYPE)
    ac_ref[...] = a_c

    drow = jax.lax.broadcasted_iota(jnp.int32, d_ref.shape, 0)
    d = jnp.where(drow < n - j * _BK, d_ref[...], jnp.bfloat16(0.0))

    @pl.when(j == 0)
    def _():
        s1_ref[...] = jnp.zeros_like(s1_ref)
        acc2_ref[...] = jnp.zeros_like(acc2_ref)

    s1_ref[...] += jnp.dot(
        a_m.astype(jnp.bfloat16), d, preferred_element_type=jnp.float32
    )

    @pl.when((j + 1) * _BK <= i * _BM)
    def _():
        s1c_blk = s1c_all[pl.ds(j * _BK, _BK), :]
        acc2_ref[...] += jnp.dot(a_c, s1c_blk, preferred_element_type=jnp.float32)

    @pl.when(j == nbj - 1)
    def _():
        row = jax.lax.broadcasted_iota(jnp.int32, s1_ref.shape, 0)
        s1c_blk = jnp.where(row < n - i * _BM, s1_ref[...], 0.0).astype(_A_DTYPE)
        s1c_ref[...] = s1c_blk
        s1c_all[pl.ds(i * _BM, _BM), :] = s1c_blk
        o2_ref[...] = acc2_ref[...]


def _hop2_kernel(
    i_arr, j_arr, first_arr, ac_ref, s1b_ref, o2_ref, s1_ref, d_ref, w_ref,
    out_ref, acc_ref, *, nbj,
):
    t = pl.program_id(0)

    @pl.when(first_arr[t] == 1)
    def _():
        acc_ref[...] = o2_ref[...]

    acc_ref[...] += jnp.dot(
        ac_ref[...], s1b_ref[...], preferred_element_type=jnp.float32
    )

    @pl.when(j_arr[t] == nbj - 1)
    def _():
        t_val = _ALPHA * d_ref[...] + ((1.0 - _ALPHA) / _K) * (
            s1_ref[...] + acc_ref[...]
        )
        out_ref[...] = jnp.maximum(
            jnp.dot(t_val, w_ref[...], preferred_element_type=jnp.float32), 0.0
        )


def kernel(adjacency_matrices, weights_matrices, data, W):
    n, c = data.shape
    f = W.shape[1]
    nbi = pl.cdiv(n, _BM)
    nbj = pl.cdiv(n, _BK)
    npr = nbi * _BM
    npc = nbj * _BK

    s1, s1c, a_c, o2 = pl.pallas_call(
        functools.partial(_hop1_kernel, nbi=nbi, nbj=nbj, n=n),
        grid=(nbi, nbj),
        in_specs=[
            pl.BlockSpec((_BM, _BK), lambda i, j: (i, j)),
            pl.BlockSpec((_BM, _BK), lambda i, j: (i, j)),
            pl.BlockSpec((_BK, c), lambda i, j: (j, 0)),
        ],
        out_specs=[
            pl.BlockSpec((_BM, c), lambda i, j: (i, 0)),
            pl.BlockSpec((_BM, c), lambda i, j: (i, 0)),
            pl.BlockSpec((_BM, _BK), lambda i, j: (i, j)),
            pl.BlockSpec((_BM, c), lambda i, j: (i, 0)),
        ],
        out_shape=[
            jax.ShapeDtypeStruct((npr, c), jnp.float32),
            jax.ShapeDtypeStruct((npr, c), _A_DTYPE),
            jax.ShapeDtypeStruct((npr, npc), _A_DTYPE),
            jax.ShapeDtypeStruct((npr, c), jnp.float32),
        ],
        scratch_shapes=[
            pltpu.VMEM((npc, c), _A_DTYPE),
            pltpu.VMEM((_BM, c), jnp.float32),
        ],
        compiler_params=pltpu.CompilerParams(
            dimension_semantics=("arbitrary", "arbitrary"),
        ),
    )(adjacency_matrices, weights_matrices, data.astype(jnp.bfloat16))

    tiles = [
        (i, j)
        for i in range(nbi)
        for j in range(nbj)
        if (j + 1) * _BK > i * _BM
    ]
    i_arr = np.array([t[0] for t in tiles], np.int32)
    j_arr = np.array([t[1] for t in tiles], np.int32)
    first_arr = np.array(
        [1 if (k == 0 or tiles[k][0] != tiles[k - 1][0]) else 0 for k in range(len(tiles))],
        np.int32,
    )

    out = pl.pallas_call(
        functools.partial(_hop2_kernel, nbj=nbj),
        grid_spec=pltpu.PrefetchScalarGridSpec(
            num_scalar_prefetch=3,
            grid=(len(tiles),),
            in_specs=[
                pl.BlockSpec((_BM, _BK), lambda t, ia, ja, fa: (ia[t], ja[t])),
                pl.BlockSpec((_BK, c), lambda t, ia, ja, fa: (ja[t], 0)),
                pl.BlockSpec((_BM, c), lambda t, ia, ja, fa: (ia[t], 0)),
                pl.BlockSpec((_BM, c), lambda t, ia, ja, fa: (ia[t], 0)),
                pl.BlockSpec((_BM, c), lambda t, ia, ja, fa: (ia[t], 0)),
                pl.BlockSpec((c, f), lambda t, ia, ja, fa: (0, 0)),
            ],
            out_specs=pl.BlockSpec((_BM, f), lambda t, ia, ja, fa: (ia[t], 0)),
            scratch_shapes=[pltpu.VMEM((_BM, c), jnp.float32)],
        ),
        out_shape=jax.ShapeDtypeStruct((n, f), jnp.float32),
        compiler_params=pltpu.CompilerParams(
            dimension_semantics=("arbitrary",),
        ),
    )(i_arr, j_arr, first_arr, a_c, s1c, o2, s1, data, W)

    return out


# speedup vs baseline: 1.7507x; 1.0315x over previous
import functools

import jax
import jax.numpy as jnp
import numpy as np
from jax.experimental import pallas as pl
from jax.experimental.pallas import tpu as pltpu

_ALPHA = 0.1
_K = 2
_BM = 1024
_BK = 2048

_A_DTYPE = jnp.float8_e4m3fn


def _hop1_kernel(
    adj_ref, w_ref, d_ref, s1_ref, s1c_ref, ac_ref, o2_ref, s1c_all, acc2_ref,
    ac_stage, ac_sem, *, nbi, nbj, n,
):
    i = pl.program_id(0)
    j = pl.program_id(1)

    a = adj_ref[...] * w_ref[...]

    col = jax.lax.broadcasted_iota(jnp.int32, a.shape, 1)
    a_m = jnp.where(col < n - j * _BK, a, 0.0)
    a_c = a_m.astype(_A_DTYPE)

    fused = (j + 1) * _BK <= i * _BM

    @pl.when(jnp.logical_not(fused))
    def _():
        @pl.when(jnp.logical_not(jnp.logical_and(i == 0, j == 0)))
        def _():
            pltpu.make_async_copy(
                ac_stage,
                ac_ref.at[pl.ds(0, _BM), pl.ds(0, _BK)],
                ac_sem,
            ).wait()

        ac_stage[...] = a_c
        pltpu.make_async_copy(
            ac_stage,
            ac_ref.at[pl.ds(i * _BM, _BM), pl.ds(j * _BK, _BK)],
            ac_sem,
        ).start()

    @pl.when(jnp.logical_and(i == nbi - 1, j == nbj - 1))
    def _():
        pltpu.make_async_copy(
            ac_stage,
            ac_ref.at[pl.ds(0, _BM), pl.ds(0, _BK)],
            ac_sem,
        ).wait()

    drow = jax.lax.broadcasted_iota(jnp.int32, d_ref.shape, 0)
    d = jnp.where(drow < n - j * _BK, d_ref[...], jnp.bfloat16(0.0))

    @pl.when(j == 0)
    def _():
        s1_ref[...] = jnp.zeros_like(s1_ref)
        acc2_ref[...] = jnp.zeros_like(acc2_ref)

    s1_ref[...] += jnp.dot(
        a_m.astype(jnp.bfloat16), d, preferred_element_type=jnp.float32
    )

    @pl.when((j + 1) * _BK <= i * _BM)
    def _():
        s1c_blk = s1c_all[pl.ds(j * _BK, _BK), :]
        acc2_ref[...] += jnp.dot(a_c, s1c_blk, preferred_element_type=jnp.float32)

    @pl.when(j == nbj - 1)
    def _():
        row = jax.lax.broadcasted_iota(jnp.int32, s1_ref.shape, 0)
        s1c_blk = jnp.where(row < n - i * _BM, s1_ref[...], 0.0).astype(_A_DTYPE)
        s1c_ref[...] = s1c_blk
        s1c_all[pl.ds(i * _BM, _BM), :] = s1c_blk
        o2_ref[...] = acc2_ref[...]


def _hop2_kernel(
    i_arr, j_arr, first_arr, ac_ref, s1b_ref, o2_ref, s1_ref, d_ref, w_ref,
    out_ref, acc_ref, *, nbj,
):
    t = pl.program_id(0)

    @pl.when(first_arr[t] == 1)
    def _():
        acc_ref[...] = o2_ref[...]

    acc_ref[...] += jnp.dot(
        ac_ref[...], s1b_ref[...], preferred_element_type=jnp.float32
    )

    @pl.when(j_arr[t] == nbj - 1)
    def _():
        t_val = _ALPHA * d_ref[...] + ((1.0 - _ALPHA) / _K) * (
            s1_ref[...] + acc_ref[...]
        )
        out_ref[...] = jnp.maximum(
            jnp.dot(t_val, w_ref[...], preferred_element_type=jnp.float32), 0.0
        )


def kernel(adjacency_matrices, weights_matrices, data, W):
    n, c = data.shape
    f = W.shape[1]
    nbi = pl.cdiv(n, _BM)
    nbj = pl.cdiv(n, _BK)
    npr = nbi * _BM
    npc = nbj * _BK

    s1, s1c, a_c, o2 = pl.pallas_call(
        functools.partial(_hop1_kernel, nbi=nbi, nbj=nbj, n=n),
        grid=(nbi, nbj),
        in_specs=[
            pl.BlockSpec((_BM, _BK), lambda i, j: (i, j)),
            pl.BlockSpec((_BM, _BK), lambda i, j: (i, j)),
            pl.BlockSpec((_BK, c), lambda i, j: (j, 0)),
        ],
        out_specs=[
            pl.BlockSpec((_BM, c), lambda i, j: (i, 0)),
            pl.BlockSpec((_BM, c), lambda i, j: (i, 0)),
            pl.BlockSpec(memory_space=pltpu.MemorySpace.HBM),
            pl.BlockSpec((_BM, c), lambda i, j: (i, 0)),
        ],
        out_shape=[
            jax.ShapeDtypeStruct((npr, c), jnp.float32),
            jax.ShapeDtypeStruct((npr, c), _A_DTYPE),
            jax.ShapeDtypeStruct((npr, npc), _A_DTYPE),
            jax.ShapeDtypeStruct((npr, c), jnp.float32),
        ],
        scratch_shapes=[
            pltpu.VMEM((npc, c), _A_DTYPE),
            pltpu.VMEM((_BM, c), jnp.float32),
            pltpu.VMEM((_BM, _BK), _A_DTYPE),
            pltpu.SemaphoreType.DMA,
        ],
        compiler_params=pltpu.CompilerParams(
            dimension_semantics=("arbitrary", "arbitrary"),
        ),
    )(adjacency_matrices, weights_matrices, data.astype(jnp.bfloat16))

    tiles = [
        (i, j)
        for i in range(nbi)
        for j in range(nbj)
        if (j + 1) * _BK > i * _BM
    ]
    i_arr = np.array([t[0] for t in tiles], np.int32)
    j_arr = np.array([t[1] for t in tiles], np.int32)
    first_arr = np.array(
        [1 if (k == 0 or tiles[k][0] != tiles[k - 1][0]) else 0 for k in range(len(tiles))],
        np.int32,
    )

    out = pl.pallas_call(
        functools.partial(_hop2_kernel, nbj=nbj),
        grid_spec=pltpu.PrefetchScalarGridSpec(
            num_scalar_prefetch=3,
            grid=(len(tiles),),
            in_specs=[
                pl.BlockSpec((_BM, _BK), lambda t, ia, ja, fa: (ia[t], ja[t])),
                pl.BlockSpec((_BK, c), lambda t, ia, ja, fa: (ja[t], 0)),
                pl.BlockSpec((_BM, c), lambda t, ia, ja, fa: (ia[t], 0)),
                pl.BlockSpec((_BM, c), lambda t, ia, ja, fa: (ia[t], 0)),
                pl.BlockSpec((_BM, c), lambda t, ia, ja, fa: (ia[t], 0)),
                pl.BlockSpec((c, f), lambda t, ia, ja, fa: (0, 0)),
            ],
            out_specs=pl.BlockSpec((_BM, f), lambda t, ia, ja, fa: (ia[t], 0)),
            scratch_shapes=[pltpu.VMEM((_BM, c), jnp.float32)],
        ),
        out_shape=jax.ShapeDtypeStruct((n, f), jnp.float32),
        compiler_params=pltpu.CompilerParams(
            dimension_semantics=("arbitrary",),
        ),
    )(i_arr, j_arr, first_arr, a_c, s1c, o2, s1, data, W)

    return out
